# baseline (speedup 1.0000x reference)
import jax
import jax.numpy as jnp
from jax import lax
from jax.experimental import pallas as pl
from jax.experimental.pallas import tpu as pltpu

_B, _T, _N, _E = 16, 96, 2048, 16
_G = _N // 128
_R = _T * _G
_EPS = 1e-5


def _body(x_hbm, gamma_ref, beta_ref, W2_ref, bg_ref,
          out_ref, msp_ref, ssp_ref, gb_ref, bb_ref, md_ref, sd_ref,
          xbuf, sem):
    b = pl.program_id(0)

    def _cp(i, slot):
        return pltpu.make_async_copy(x_hbm.at[i], xbuf.at[slot], sem.at[slot])

    @pl.when(b == 0)
    def _():
        _cp(0, 0).start()

    @pl.when(b + 1 < _B)
    def _():
        _cp(b + 1, (b + 1) % 2).start()

    _cp(b, b % 2).wait()
    x4 = xbuf[b % 2].reshape(_T, _G, 128)
    m1 = jnp.mean(x4, axis=0)
    m2 = jnp.mean(x4 * x4, axis=0)
    mu_d = jnp.mean(m1)
    var_d = jnp.mean(m2) - mu_d * mu_d
    std_d = jnp.sqrt(var_d + _EPS)
    mean_sp = (m1 - mu_d) / std_d
    var_sp = (m2 - m1 * m1) / (var_d + _EPS)
    std_sp = jnp.sqrt(var_sp + _EPS)
    s = 1.0 / (std_d * std_sp)

    xn = (x4 - m1[None]) * s[None]
    xnr = xn.reshape(_R, 128)

    sc = jnp.dot(W2_ref[...], xnr,
                 preferred_element_type=jnp.float32)
    sc = sc.reshape(_G, _E, 128) + bg_ref[...].reshape(1, _E, 1)
    mx = jnp.max(sc, axis=1, keepdims=True)
    ii = lax.broadcasted_iota(jnp.int32, (_G, _E, 128), 1)
    cand = jnp.where(sc == mx, ii, _E)
    idx = jnp.min(cand, axis=1)

    gam = gamma_ref[...]
    bet = beta_ref[...]
    gsel = jnp.full((_G, 128), gam[0, 0])
    bsel = jnp.full((_G, 128), bet[0, 0])
    for e in range(1, _E):
        pe = idx == e
        gsel = jnp.where(pe, gam[e, 0], gsel)
        bsel = jnp.where(pe, bet[e, 0], bsel)

    out4 = xn * gsel[None] + bsel[None]
    out_ref[0] = out4.reshape(_R, 128)
    msp_ref[0] = mean_sp
    ssp_ref[0] = std_sp
    gb_ref[0] = gsel
    bb_ref[0] = bsel
    md_ref[...] = jnp.full((1, 1, 1), mu_d)
    sd_ref[...] = jnp.full((1, 1, 1), std_d)


def kernel(x, gamma, beta, Wg, bg):
    x2 = pltpu.with_memory_space_constraint(
        x.reshape(_B, _R, 128), pltpu.MemorySpace.HBM)
    bg2 = bg.reshape(_E, 1)
    W2 = jnp.einsum('et,gh->geth', Wg,
                    jnp.eye(_G, dtype=Wg.dtype)).reshape(_G * _E, _R)
    rep = pl.BlockSpec((_E, 1), lambda b: (0, 0))
    wspec = pl.BlockSpec((_G * _E, _R), lambda b: (0, 0))
    out, msp, ssp, gb, bb, md, sd = pl.pallas_call(
        _body,
        grid=(_B,),
        in_specs=[
            pl.BlockSpec(memory_space=pl.ANY),
            rep, rep, wspec, rep,
        ],
        scratch_shapes=[
            pltpu.VMEM((2, _R, 128), jnp.float32),
            pltpu.SemaphoreType.DMA((2,)),
        ],
        out_specs=[
            pl.BlockSpec((1, _R, 128), lambda b: (b, 0, 0)),
            pl.BlockSpec((1, _G, 128), lambda b: (b, 0, 0)),
            pl.BlockSpec((1, _G, 128), lambda b: (b, 0, 0)),
            pl.BlockSpec((1, _G, 128), lambda b: (b, 0, 0)),
            pl.BlockSpec((1, _G, 128), lambda b: (b, 0, 0)),
            pl.BlockSpec((1, 1, 1), lambda b: (b, 0, 0)),
            pl.BlockSpec((1, 1, 1), lambda b: (b, 0, 0)),
        ],
        out_shape=[
            jax.ShapeDtypeStruct((_B, _R, 128), jnp.float32),
            jax.ShapeDtypeStruct((_B, _G, 128), jnp.float32),
            jax.ShapeDtypeStruct((_B, _G, 128), jnp.float32),
            jax.ShapeDtypeStruct((_B, _G, 128), jnp.float32),
            jax.ShapeDtypeStruct((_B, _G, 128), jnp.float32),
            jax.ShapeDtypeStruct((_B, 1, 1), jnp.float32),
            jax.ShapeDtypeStruct((_B, 1, 1), jnp.float32),
        ],
    )(x2, gamma, beta, W2, bg2)
    return (out.reshape(_B, _T, _N, 1),
            gb.reshape(_B, 1, _N, 1),
            bb.reshape(_B, 1, _N, 1),
            msp.reshape(_B, 1, _N, 1),
            ssp.reshape(_B, 1, _N, 1),
            md.reshape(_B, 1, 1, 1),
            sd.reshape(_B, 1, 1, 1))


# speedup vs baseline: 1.4942x; 1.4942x over previous
import jax
import jax.numpy as jnp
from jax import lax
from jax.experimental import pallas as pl
from jax.experimental.pallas import tpu as pltpu

_B, _T, _N, _E = 16, 96, 2048, 16
_G = _N // 128
_R = _T * _G
_EPS = 1e-5


def _body(x_hbm, gamma_ref, beta_ref, W2_ref, bg_ref,
          out_ref, msp_ref, ssp_ref, gb_ref, bb_ref, md_ref, sd_ref,
          xbuf, sem):
    b = pl.program_id(0)

    def _cp(i, slot):
        return pltpu.make_async_copy(x_hbm.at[i], xbuf.at[slot], sem.at[slot])

    @pl.when(b == 0)
    def _():
        _cp(0, 0).start()

    @pl.when(b + 1 < _B)
    def _():
        _cp(b + 1, (b + 1) % 2).start()

    _cp(b, b % 2).wait()
    x4 = xbuf[b % 2].reshape(_T, _G, 128)
    m1 = jnp.mean(x4, axis=0)
    m2 = jnp.mean(x4 * x4, axis=0)
    mu_d = jnp.mean(m1)
    var_d = jnp.mean(m2) - mu_d * mu_d
    std_d = jnp.sqrt(var_d + _EPS)
    mean_sp = (m1 - mu_d) / std_d
    var_sp = (m2 - m1 * m1) / (var_d + _EPS)
    std_sp = jnp.sqrt(var_sp + _EPS)
    s = 1.0 / (std_d * std_sp)

    xn = (x4 - m1[None]) * s[None]
    xnr = xn.reshape(_R, 128)

    sc = (jnp.dot(W2_ref[...], xnr, preferred_element_type=jnp.float32)
          + bg_ref[...]).reshape(_E, _G, 128)
    best = sc[0]
    idx = jnp.zeros((_G, 128), jnp.int32)
    for e in range(1, _E):
        upd = sc[e] > best
        best = jnp.where(upd, sc[e], best)
        idx = jnp.where(upd, e, idx)

    gam = gamma_ref[...]
    bet = beta_ref[...]
    gsel = jnp.full((_G, 128), gam[0, 0])
    bsel = jnp.full((_G, 128), bet[0, 0])
    for e in range(1, _E):
        pe = idx == e
        gsel = jnp.where(pe, gam[e, 0], gsel)
        bsel = jnp.where(pe, bet[e, 0], bsel)

    out4 = xn * gsel[None] + bsel[None]
    out_ref[0] = out4.reshape(_R, 128)
    msp_ref[0] = mean_sp
    ssp_ref[0] = std_sp
    gb_ref[0] = gsel
    bb_ref[0] = bsel
    md_ref[...] = jnp.full((1, 1, 1), mu_d)
    sd_ref[...] = jnp.full((1, 1, 1), std_d)


def kernel(x, gamma, beta, Wg, bg):
    x2 = pltpu.with_memory_space_constraint(
        x.reshape(_B, _R, 128), pltpu.MemorySpace.HBM)
    bg2 = jnp.repeat(bg, _G).reshape(_E * _G, 1)
    wrep = jnp.repeat(Wg, _G, axis=1)
    wbig = jnp.broadcast_to(wrep[:, None, :],
                            (_E, _G, _R)).reshape(_E * _G, _R)
    rr = lax.broadcasted_iota(jnp.int32, (_E * _G, _R), 0)
    cc = lax.broadcasted_iota(jnp.int32, (_E * _G, _R), 1)
    W2 = jnp.where(rr % _G == cc % _G, wbig, 0.0)
    rep = pl.BlockSpec((_E, 1), lambda b: (0, 0))
    wspec = pl.BlockSpec((_E * _G, _R), lambda b: (0, 0))
    out, msp, ssp, gb, bb, md, sd = pl.pallas_call(
        _body,
        grid=(_B,),
        in_specs=[
            pl.BlockSpec(memory_space=pl.ANY),
            rep, rep, wspec,
            pl.BlockSpec((_E * _G, 1), lambda b: (0, 0)),
        ],
        scratch_shapes=[
            pltpu.VMEM((2, _R, 128), jnp.float32),
            pltpu.SemaphoreType.DMA((2,)),
        ],
        out_specs=[
            pl.BlockSpec((1, _R, 128), lambda b: (b, 0, 0)),
            pl.BlockSpec((1, _G, 128), lambda b: (b, 0, 0)),
            pl.BlockSpec((1, _G, 128), lambda b: (b, 0, 0)),
            pl.BlockSpec((1, _G, 128), lambda b: (b, 0, 0)),
            pl.BlockSpec((1, _G, 128), lambda b: (b, 0, 0)),
            pl.BlockSpec((1, 1, 1), lambda b: (b, 0, 0)),
            pl.BlockSpec((1, 1, 1), lambda b: (b, 0, 0)),
        ],
        out_shape=[
            jax.ShapeDtypeStruct((_B, _R, 128), jnp.float32),
            jax.ShapeDtypeStruct((_B, _G, 128), jnp.float32),
            jax.ShapeDtypeStruct((_B, _G, 128), jnp.float32),
            jax.ShapeDtypeStruct((_B, _G, 128), jnp.float32),
            jax.ShapeDtypeStruct((_B, _G, 128), jnp.float32),
            jax.ShapeDtypeStruct((_B, 1, 1), jnp.float32),
            jax.ShapeDtypeStruct((_B, 1, 1), jnp.float32),
        ],
    )(x2, gamma, beta, W2, bg2)
    return (out.reshape(_B, _T, _N, 1),
            gb.reshape(_B, 1, _N, 1),
            bb.reshape(_B, 1, _N, 1),
            msp.reshape(_B, 1, _N, 1),
            ssp.reshape(_B, 1, _N, 1),
            md.reshape(_B, 1, 1, 1),
            sd.reshape(_B, 1, 1, 1))


# speedup vs baseline: 1.5541x; 1.0400x over previous
import jax
import jax.numpy as jnp
from jax import lax
from jax.experimental import pallas as pl
from jax.experimental.pallas import tpu as pltpu

_B, _T, _N, _E = 16, 96, 2048, 16
_G = _N // 128
_R = _T * _G
_EPS = 1e-5


def _body(x_hbm, aux_ref,
          out_ref, msp_ref, ssp_ref, gb_ref, bb_ref, md_ref, sd_ref,
          xbuf, sem):
    b = pl.program_id(0)

    def _cp(i, slot):
        return pltpu.make_async_copy(x_hbm.at[i], xbuf.at[slot], sem.at[slot])

    @pl.when(b == 0)
    def _():
        _cp(0, 0).start()

    @pl.when(b + 1 < _B)
    def _():
        _cp(b + 1, (b + 1) % 2).start()

    _cp(b, b % 2).wait()
    x4 = xbuf[b % 2].reshape(_T, _G, 128)
    m1 = jnp.mean(x4, axis=0)
    m2 = jnp.mean(x4 * x4, axis=0)
    mu_d = jnp.mean(m1)
    var_d = jnp.mean(m2) - mu_d * mu_d
    std_d = jnp.sqrt(var_d + _EPS)
    mean_sp = (m1 - mu_d) / std_d
    var_sp = (m2 - m1 * m1) / (var_d + _EPS)
    std_sp = jnp.sqrt(var_sp + _EPS)
    s = 1.0 / (std_d * std_sp)

    xn = (x4 - m1[None]) * s[None]
    xnr = xn.reshape(_R, 128)

    sc = (jnp.dot(aux_ref[:, :_R], xnr, preferred_element_type=jnp.float32)
          + aux_ref[:, _R:_R + 1]).reshape(_E, _G, 128)
    best = sc[0]
    idx = jnp.zeros((_G, 128), jnp.int32)
    for e in range(1, _E):
        upd = sc[e] > best
        best = jnp.where(upd, sc[e], best)
        idx = jnp.where(upd, e, idx)

    gsel = jnp.full((_G, 128), aux_ref[0, _R + 1])
    bsel = jnp.full((_G, 128), aux_ref[0, _R + 2])
    for e in range(1, _E):
        pe = idx == e
        gsel = jnp.where(pe, aux_ref[e * _G, _R + 1], gsel)
        bsel = jnp.where(pe, aux_ref[e * _G, _R + 2], bsel)

    out4 = xn * gsel[None] + bsel[None]
    out_ref[0] = out4.reshape(_R, 128)
    msp_ref[0] = mean_sp
    ssp_ref[0] = std_sp
    gb_ref[0] = gsel
    bb_ref[0] = bsel
    md_ref[...] = jnp.full((1, 1, 1), mu_d)
    sd_ref[...] = jnp.full((1, 1, 1), std_d)


def kernel(x, gamma, beta, Wg, bg):
    x2 = pltpu.with_memory_space_constraint(
        x.reshape(_B, _R, 128), pltpu.MemorySpace.HBM)
    bg2 = jnp.repeat(bg, _G).reshape(_E * _G, 1)
    wrep = jnp.repeat(Wg, _G, axis=1)
    wbig = jnp.broadcast_to(wrep[:, None, :],
                            (_E, _G, _R)).reshape(_E * _G, _R)
    rr = lax.broadcasted_iota(jnp.int32, (_E * _G, _R), 0)
    cc = lax.broadcasted_iota(jnp.int32, (_E * _G, _R), 1)
    W2 = jnp.where(rr % _G == cc % _G, wbig, 0.0)
    gcol = jnp.repeat(gamma[:, 0], _G).reshape(_E * _G, 1)
    bcol = jnp.repeat(beta[:, 0], _G).reshape(_E * _G, 1)
    aux = jnp.concatenate([W2, bg2, gcol, bcol], axis=1)
    out, msp, ssp, gb, bb, md, sd = pl.pallas_call(
        _body,
        grid=(_B,),
        in_specs=[
            pl.BlockSpec(memory_space=pl.ANY),
            pl.BlockSpec((_E * _G, _R + 3), lambda b: (0, 0)),
        ],
        scratch_shapes=[
            pltpu.VMEM((2, _R, 128), jnp.float32),
            pltpu.SemaphoreType.DMA((2,)),
        ],
        out_specs=[
            pl.BlockSpec((1, _R, 128), lambda b: (b, 0, 0)),
            pl.BlockSpec((1, _G, 128), lambda b: (b, 0, 0)),
            pl.BlockSpec((1, _G, 128), lambda b: (b, 0, 0)),
            pl.BlockSpec((1, _G, 128), lambda b: (b, 0, 0)),
            pl.BlockSpec((1, _G, 128), lambda b: (b, 0, 0)),
            pl.BlockSpec((1, 1, 1), lambda b: (b, 0, 0)),
            pl.BlockSpec((1, 1, 1), lambda b: (b, 0, 0)),
        ],
        out_shape=[
            jax.ShapeDtypeStruct((_B, _R, 128), jnp.float32),
            jax.ShapeDtypeStruct((_B, _G, 128), jnp.float32),
            jax.ShapeDtypeStruct((_B, _G, 128), jnp.float32),
            jax.ShapeDtypeStruct((_B, _G, 128), jnp.float32),
            jax.ShapeDtypeStruct((_B, _G, 128), jnp.float32),
            jax.ShapeDtypeStruct((_B, 1, 1), jnp.float32),
            jax.ShapeDtypeStruct((_B, 1, 1), jnp.float32),
        ],
    )(x2, aux)
    return (out.reshape(_B, _T, _N, 1),
            gb.reshape(_B, 1, _N, 1),
            bb.reshape(_B, 1, _N, 1),
            msp.reshape(_B, 1, _N, 1),
            ssp.reshape(_B, 1, _N, 1),
            md.reshape(_B, 1, 1, 1),
            sd.reshape(_B, 1, 1, 1))


# speedup vs baseline: 1.6214x; 1.0433x over previous
import jax
import jax.numpy as jnp
from jax import lax
from jax.experimental import pallas as pl
from jax.experimental.pallas import tpu as pltpu

_B, _T, _N, _E = 16, 96, 2048, 16
_G = _N // 128
_R = _T * _G
_EPS = 1e-5


def _body(x_hbm, aux_ref,
          out_ref, msp_ref, ssp_ref, gb_ref, bb_ref, md_ref, sd_ref,
          xbuf, sem):
    b = pl.program_id(0)

    def _cp(i, slot):
        return pltpu.make_async_copy(x_hbm.at[i], xbuf.at[slot], sem.at[slot])

    @pl.when(b == 0)
    def _():
        _cp(0, 0).start()

    @pl.when(b + 1 < _B)
    def _():
        _cp(b + 1, (b + 1) % 2).start()

    _cp(b, b % 2).wait()
    x4 = xbuf[b % 2].reshape(_T, _G, 128)
    m1 = jnp.mean(x4, axis=0)
    m2 = jnp.mean(x4 * x4, axis=0)
    mu_d = jnp.mean(m1)
    var_d = jnp.mean(m2) - mu_d * mu_d
    std_d = jnp.sqrt(var_d + _EPS)
    mean_sp = (m1 - mu_d) / std_d
    var_sp = (m2 - m1 * m1) / (var_d + _EPS)
    std_sp = jnp.sqrt(var_sp + _EPS)
    s = 1.0 / (std_d * std_sp)

    gr = jnp.dot(aux_ref[:, :_R], xbuf[b % 2],
                 preferred_element_type=jnp.float32).reshape(_E, _G, 128)
    best = (gr[0] - aux_ref[0, _R + 3] * m1) * s + aux_ref[0, _R]
    idx = jnp.zeros((_G, 128), jnp.int32)
    for e in range(1, _E):
        se = (gr[e] - aux_ref[e * _G, _R + 3] * m1) * s + aux_ref[e * _G, _R]
        upd = se > best
        best = jnp.where(upd, se, best)
        idx = jnp.where(upd, e, idx)

    gsel = jnp.full((_G, 128), aux_ref[0, _R + 1])
    bsel = jnp.full((_G, 128), aux_ref[0, _R + 2])
    for e in range(1, _E):
        pe = idx == e
        gsel = jnp.where(pe, aux_ref[e * _G, _R + 1], gsel)
        bsel = jnp.where(pe, aux_ref[e * _G, _R + 2], bsel)

    a = s * gsel
    out4 = (x4 - m1[None]) * a[None] + bsel[None]
    out_ref[0] = out4.reshape(_R, 128)
    msp_ref[0] = mean_sp
    ssp_ref[0] = std_sp
    gb_ref[0] = gsel
    bb_ref[0] = bsel
    md_ref[...] = jnp.full((1, 1, 1), mu_d)
    sd_ref[...] = jnp.full((1, 1, 1), std_d)


def kernel(x, gamma, beta, Wg, bg):
    x2 = pltpu.with_memory_space_constraint(
        x.reshape(_B, _R, 128), pltpu.MemorySpace.HBM)
    bg2 = jnp.repeat(bg, _G).reshape(_E * _G, 1)
    wrep = jnp.repeat(Wg, _G, axis=1)
    wbig = jnp.broadcast_to(wrep[:, None, :],
                            (_E, _G, _R)).reshape(_E * _G, _R)
    rr = lax.broadcasted_iota(jnp.int32, (_E * _G, _R), 0)
    cc = lax.broadcasted_iota(jnp.int32, (_E * _G, _R), 1)
    W2 = jnp.where(rr % _G == cc % _G, wbig, 0.0)
    gcol = jnp.repeat(gamma[:, 0], _G).reshape(_E * _G, 1)
    bcol = jnp.repeat(beta[:, 0], _G).reshape(_E * _G, 1)
    w1col = jnp.repeat(jnp.sum(Wg, axis=1), _G).reshape(_E * _G, 1)
    aux = jnp.concatenate([W2, bg2, gcol, bcol, w1col], axis=1)
    out, msp, ssp, gb, bb, md, sd = pl.pallas_call(
        _body,
        grid=(_B,),
        in_specs=[
            pl.BlockSpec(memory_space=pl.ANY),
            pl.BlockSpec((_E * _G, _R + 4), lambda b: (0, 0)),
        ],
        scratch_shapes=[
            pltpu.VMEM((2, _R, 128), jnp.float32),
            pltpu.SemaphoreType.DMA((2,)),
        ],
        out_specs=[
            pl.BlockSpec((1, _R, 128), lambda b: (b, 0, 0)),
            pl.BlockSpec((1, _G, 128), lambda b: (b, 0, 0)),
            pl.BlockSpec((1, _G, 128), lambda b: (b, 0, 0)),
            pl.BlockSpec((1, _G, 128), lambda b: (b, 0, 0)),
            pl.BlockSpec((1, _G, 128), lambda b: (b, 0, 0)),
            pl.BlockSpec((1, 1, 1), lambda b: (b, 0, 0)),
            pl.BlockSpec((1, 1, 1), lambda b: (b, 0, 0)),
        ],
        out_shape=[
            jax.ShapeDtypeStruct((_B, _R, 128), jnp.float32),
            jax.ShapeDtypeStruct((_B, _G, 128), jnp.float32),
            jax.ShapeDtypeStruct((_B, _G, 128), jnp.float32),
            jax.ShapeDtypeStruct((_B, _G, 128), jnp.float32),
            jax.ShapeDtypeStruct((_B, _G, 128), jnp.float32),
            jax.ShapeDtypeStruct((_B, 1, 1), jnp.float32),
            jax.ShapeDtypeStruct((_B, 1, 1), jnp.float32),
        ],
    )(x2, aux)
    return (out.reshape(_B, _T, _N, 1),
            gb.reshape(_B, 1, _N, 1),
            bb.reshape(_B, 1, _N, 1),
            msp.reshape(_B, 1, _N, 1),
            ssp.reshape(_B, 1, _N, 1),
            md.reshape(_B, 1, 1, 1),
            sd.reshape(_B, 1, 1, 1))


# speedup vs baseline: 1.8891x; 1.1651x over previous
import jax
import jax.numpy as jnp
from jax import lax
from jax.experimental import pallas as pl
from jax.experimental.pallas import tpu as pltpu

_B, _T, _N, _E = 16, 96, 2048, 16
_G = _N // 128
_R = _T * _G
_EPS = 1e-5


def _body(x_hbm, aux_ref,
          out_ref, msp_ref, ssp_ref, gb_ref, bb_ref, md_ref, sd_ref,
          xbuf, sem):
    b = pl.program_id(0)
    _S, _C = 3, 4
    _RC = _R // _C

    def _start(i, slot):
        for c in range(_C):
            pltpu.make_async_copy(x_hbm.at[i, pl.ds(c * _RC, _RC)],
                                  xbuf.at[slot, pl.ds(c * _RC, _RC)],
                                  sem.at[slot, c]).start()

    def _wait(i, slot):
        for c in range(_C):
            pltpu.make_async_copy(x_hbm.at[i, pl.ds(c * _RC, _RC)],
                                  xbuf.at[slot, pl.ds(c * _RC, _RC)],
                                  sem.at[slot, c]).wait()

    @pl.when(b == 0)
    def _():
        _start(0, 0)
        _start(1, 1)

    @pl.when(b + 2 < _B)
    def _():
        _start(b + 2, (b + 2) % _S)

    _wait(b, b % _S)
    x4 = xbuf[b % _S].reshape(_T, _G, 128)
    m1 = jnp.mean(x4, axis=0)
    m2 = jnp.mean(x4 * x4, axis=0)
    mu_d = jnp.mean(m1)
    var_d = jnp.mean(m2) - mu_d * mu_d
    std_d = jnp.sqrt(var_d + _EPS)
    mean_sp = (m1 - mu_d) / std_d
    var_sp = (m2 - m1 * m1) / (var_d + _EPS)
    std_sp = jnp.sqrt(var_sp + _EPS)
    s = 1.0 / (std_d * std_sp)

    gr = jnp.dot(aux_ref[:, :_R], xbuf[b % _S],
                 preferred_element_type=jnp.float32).reshape(_E, _G, 128)
    best = (gr[0] - aux_ref[0, _R + 3] * m1) * s + aux_ref[0, _R]
    idx = jnp.zeros((_G, 128), jnp.int32)
    for e in range(1, _E):
        se = (gr[e] - aux_ref[e * _G, _R + 3] * m1) * s + aux_ref[e * _G, _R]
        upd = se > best
        best = jnp.where(upd, se, best)
        idx = jnp.where(upd, e, idx)

    gsel = jnp.full((_G, 128), aux_ref[0, _R + 1])
    bsel = jnp.full((_G, 128), aux_ref[0, _R + 2])
    for e in range(1, _E):
        pe = idx == e
        gsel = jnp.where(pe, aux_ref[e * _G, _R + 1], gsel)
        bsel = jnp.where(pe, aux_ref[e * _G, _R + 2], bsel)

    a = s * gsel
    out4 = (x4 - m1[None]) * a[None] + bsel[None]
    out_ref[0] = out4.reshape(_R, 128)
    msp_ref[0] = mean_sp
    ssp_ref[0] = std_sp
    gb_ref[0] = gsel
    bb_ref[0] = bsel
    md_ref[...] = jnp.full((1, 1, 1), mu_d)
    sd_ref[...] = jnp.full((1, 1, 1), std_d)


def kernel(x, gamma, beta, Wg, bg):
    x2 = pltpu.with_memory_space_constraint(
        x.reshape(_B, _R, 128), pltpu.MemorySpace.HBM)
    bg2 = jnp.repeat(bg, _G).reshape(_E * _G, 1)
    wrep = jnp.repeat(Wg, _G, axis=1)
    wbig = jnp.broadcast_to(wrep[:, None, :],
                            (_E, _G, _R)).reshape(_E * _G, _R)
    rr = lax.broadcasted_iota(jnp.int32, (_E * _G, _R), 0)
    cc = lax.broadcasted_iota(jnp.int32, (_E * _G, _R), 1)
    W2 = jnp.where(rr % _G == cc % _G, wbig, 0.0)
    gcol = jnp.repeat(gamma[:, 0], _G).reshape(_E * _G, 1)
    bcol = jnp.repeat(beta[:, 0], _G).reshape(_E * _G, 1)
    w1col = jnp.repeat(jnp.sum(Wg, axis=1), _G).reshape(_E * _G, 1)
    aux = jnp.concatenate([W2, bg2, gcol, bcol, w1col], axis=1)
    out, msp, ssp, gb, bb, md, sd = pl.pallas_call(
        _body,
        grid=(_B,),
        in_specs=[
            pl.BlockSpec(memory_space=pl.ANY),
            pl.BlockSpec((_E * _G, _R + 4), lambda b: (0, 0)),
        ],
        scratch_shapes=[
            pltpu.VMEM((3, _R, 128), jnp.float32),
            pltpu.SemaphoreType.DMA((3, 4)),
        ],
        out_specs=[
            pl.BlockSpec((1, _R, 128), lambda b: (b, 0, 0)),
            pl.BlockSpec((1, _G, 128), lambda b: (b, 0, 0)),
            pl.BlockSpec((1, _G, 128), lambda b: (b, 0, 0)),
            pl.BlockSpec((1, _G, 128), lambda b: (b, 0, 0)),
            pl.BlockSpec((1, _G, 128), lambda b: (b, 0, 0)),
            pl.BlockSpec((1, 1, 1), lambda b: (b, 0, 0)),
            pl.BlockSpec((1, 1, 1), lambda b: (b, 0, 0)),
        ],
        out_shape=[
            jax.ShapeDtypeStruct((_B, _R, 128), jnp.float32),
            jax.ShapeDtypeStruct((_B, _G, 128), jnp.float32),
            jax.ShapeDtypeStruct((_B, _G, 128), jnp.float32),
            jax.ShapeDtypeStruct((_B, _G, 128), jnp.float32),
            jax.ShapeDtypeStruct((_B, _G, 128), jnp.float32),
            jax.ShapeDtypeStruct((_B, 1, 1), jnp.float32),
            jax.ShapeDtypeStruct((_B, 1, 1), jnp.float32),
        ],
    )(x2, aux)
    return (out.reshape(_B, _T, _N, 1),
            gb.reshape(_B, 1, _N, 1),
            bb.reshape(_B, 1, _N, 1),
            msp.reshape(_B, 1, _N, 1),
            ssp.reshape(_B, 1, _N, 1),
            md.reshape(_B, 1, 1, 1),
            sd.reshape(_B, 1, 1, 1))
